# baseline (speedup 1.0000x reference)
import functools

import jax
import jax.numpy as jnp
from jax import lax
from jax.experimental import pallas as pl
from jax.experimental.pallas import tpu as pltpu
from jax.experimental.pallas import tpu_sc as plsc

IN_DIM = 256
OUT_DIM = 128
BATCH = 16384

K_MAX = 32
S_W = 2 * IN_DIM + 16
ZERO_COL = 2 * IN_DIM
N_STRIPE = OUT_DIM // 16

NC = 2
NS = 16
NW = NC * NS
ROWS_PER_CHUNK = 16
CHUNKS_PER_W = BATCH // (NW * ROWS_PER_CHUNK)


def _build_edges(mat):
    m = mat.astype(jnp.int32)
    iszero = (m == 0)
    order = jnp.argsort(iszero, axis=1, stable=True).astype(jnp.int32)
    cols = order[:, :K_MAX]
    t = jnp.take_along_axis(m, cols, axis=1)
    sidx = jnp.where(t == 0, ZERO_COL, cols + IN_DIM * (t == 2).astype(jnp.int32))
    return sidx.reshape(N_STRIPE, 16, K_MAX).transpose(0, 2, 1)


def _sc_body(x_hbm, e_hbm, out_hbm, xb, s, eb, ob, sem):
    del sem
    wid = lax.axis_index("s") * NC + lax.axis_index("c")
    pltpu.sync_copy(e_hbm, eb)

    def chunk_body(j, carry):
        base = (wid * CHUNKS_PER_W + j) * ROWS_PER_CHUNK
        pltpu.sync_copy(x_hbm.at[pl.ds(base, ROWS_PER_CHUNK)], xb)

        def srow_body(b, c2):
            for c in range(IN_DIM // 16):
                v = xb[b, pl.ds(c * 16, 16)]
                s[b, pl.ds(c * 16, 16)] = v * 0.5
                e = jnp.exp(v)
                s[b, pl.ds(IN_DIM + c * 16, 16)] = 1.0 - 2.0 / (e + 1.0)
            s[b, pl.ds(2 * IN_DIM, 16)] = jnp.zeros((16,), jnp.float32)
            for g in range(OUT_DIM // 16):
                ob[b, pl.ds(g * 16, 16)] = jnp.zeros((16,), jnp.float32)
            return c2

        lax.fori_loop(0, ROWS_PER_CHUNK, srow_body, 0)

        for r in range(N_STRIPE):
            def k_body(k, c2, r=r):
                colv = eb[r, k, :]
                for b in range(ROWS_PER_CHUNK):
                    bv = jnp.full((16,), b, jnp.int32)
                    v = plsc.load_gather(s, [bv, colv])
                    plsc.addupdate(ob.at[b, pl.ds(r * 16, 16)], v)
                return c2

            lax.fori_loop(0, K_MAX, k_body, 0)

        pltpu.sync_copy(ob, out_hbm.at[pl.ds(base, ROWS_PER_CHUNK)])
        return carry

    lax.fori_loop(0, CHUNKS_PER_W, chunk_body, 0)


@functools.partial(jax.jit, static_argnames=())
def kernel(x, mat):
    e = _build_edges(mat)
    mesh = plsc.VectorSubcoreMesh(core_axis_name="c", subcore_axis_name="s")
    f = functools.partial(
        pl.kernel,
        out_type=jax.ShapeDtypeStruct((BATCH, OUT_DIM), jnp.float32),
        mesh=mesh,
        scratch_types=[
            pltpu.VMEM((ROWS_PER_CHUNK, IN_DIM), jnp.float32),
            pltpu.VMEM((ROWS_PER_CHUNK, S_W), jnp.float32),
            pltpu.VMEM((N_STRIPE, K_MAX, 16), jnp.int32),
            pltpu.VMEM((ROWS_PER_CHUNK, OUT_DIM), jnp.float32),
            pltpu.SemaphoreType.DMA,
        ],
        compiler_params=pltpu.CompilerParams(
            use_tc_tiling_on_sc=False, needs_layout_passes=False),
    )(_sc_body)
    return f(x, e)


# speedup vs baseline: 2.3648x; 2.3648x over previous
import functools

import jax
import jax.numpy as jnp
from jax import lax
from jax.experimental import pallas as pl
from jax.experimental.pallas import tpu as pltpu
from jax.experimental.pallas import tpu_sc as plsc

IN_DIM = 256
OUT_DIM = 128
BATCH = 16384

K_MAX = 32
S_W = 2 * IN_DIM + 16
ZERO_COL = 2 * IN_DIM
N_STRIPE = OUT_DIM // 16

NC = 2
NS = 16
NW = NC * NS
ROWS_PER_CHUNK = 16
CHUNKS_PER_W = BATCH // (NW * ROWS_PER_CHUNK)


def _build_edges(mat):
    m = mat.astype(jnp.int32)
    iszero = (m == 0)
    order = jnp.argsort(iszero, axis=1, stable=True).astype(jnp.int32)
    cols = order[:, :K_MAX]
    t = jnp.take_along_axis(m, cols, axis=1)
    sidx = jnp.where(t == 0, ZERO_COL, cols + IN_DIM * (t == 2).astype(jnp.int32))
    return sidx.reshape(N_STRIPE, 16, K_MAX).transpose(0, 2, 1)


def _sc_body(x_hbm, e_hbm, out_hbm, xb, s, eb, ob, sem):
    del sem
    wid = lax.axis_index("s") * NC + lax.axis_index("c")
    pltpu.sync_copy(e_hbm, eb)

    def chunk_body(j, carry):
        base = (wid * CHUNKS_PER_W + j) * ROWS_PER_CHUNK
        pltpu.sync_copy(x_hbm.at[pl.ds(base, ROWS_PER_CHUNK)], xb)

        def srow_body(b, c2):
            for c in range(IN_DIM // 16):
                v = xb[b, pl.ds(c * 16, 16)]
                s[pl.ds(b * S_W + c * 16, 16)] = v * 0.5
                e = jnp.exp(v)
                s[pl.ds(b * S_W + IN_DIM + c * 16, 16)] = 1.0 - 2.0 / (e + 1.0)
            s[pl.ds(b * S_W + 2 * IN_DIM, 16)] = jnp.zeros((16,), jnp.float32)
            return c2

        lax.fori_loop(0, ROWS_PER_CHUNK, srow_body, 0)

        zero = jnp.zeros((16,), jnp.float32)
        for r in range(N_STRIPE):
            def k_body(k, accs, r=r):
                colv = eb[r, k, :]
                return tuple(
                    accs[b] + plsc.load_gather(s, [colv + b * S_W])
                    for b in range(ROWS_PER_CHUNK)
                )

            accs = lax.fori_loop(
                0, K_MAX, k_body, tuple(zero for _ in range(ROWS_PER_CHUNK)))
            for b in range(ROWS_PER_CHUNK):
                ob[b, pl.ds(r * 16, 16)] = accs[b]

        pltpu.sync_copy(ob, out_hbm.at[pl.ds(base, ROWS_PER_CHUNK)])
        return carry

    lax.fori_loop(0, CHUNKS_PER_W, chunk_body, 0)


@functools.partial(jax.jit, static_argnames=())
def kernel(x, mat):
    e = _build_edges(mat)
    mesh = plsc.VectorSubcoreMesh(core_axis_name="c", subcore_axis_name="s")
    f = functools.partial(
        pl.kernel,
        out_type=jax.ShapeDtypeStruct((BATCH, OUT_DIM), jnp.float32),
        mesh=mesh,
        scratch_types=[
            pltpu.VMEM((ROWS_PER_CHUNK, IN_DIM), jnp.float32),
            pltpu.VMEM((ROWS_PER_CHUNK * S_W,), jnp.float32),
            pltpu.VMEM((N_STRIPE, K_MAX, 16), jnp.int32),
            pltpu.VMEM((ROWS_PER_CHUNK, OUT_DIM), jnp.float32),
            pltpu.SemaphoreType.DMA,
        ],
        compiler_params=pltpu.CompilerParams(
            use_tc_tiling_on_sc=False, needs_layout_passes=False),
    )(_sc_body)
    return f(x, e)


# speedup vs baseline: 2.6383x; 1.1157x over previous
import functools

import jax
import jax.numpy as jnp
from jax import lax
from jax.experimental import pallas as pl
from jax.experimental.pallas import tpu as pltpu
from jax.experimental.pallas import tpu_sc as plsc

IN_DIM = 256
OUT_DIM = 128
BATCH = 16384

K_MAX = 32
S_W = 2 * IN_DIM + 16
ZERO_COL = 2 * IN_DIM
N_STRIPE = OUT_DIM // 16

NC = 2
NS = 16
NW = NC * NS
ROWS_PER_CHUNK = 16
CHUNKS_PER_W = BATCH // (NW * ROWS_PER_CHUNK)


def _build_edges(mat):
    m = mat.astype(jnp.int32)
    nnz = jnp.sum(m != 0, axis=1)
    perm = jnp.argsort(-nnz).astype(jnp.int32)
    ms = m[perm]
    iszero = (ms == 0)
    order = jnp.argsort(iszero, axis=1, stable=True).astype(jnp.int32)
    cols = order[:, :K_MAX]
    t = jnp.take_along_axis(ms, cols, axis=1)
    sidx = jnp.where(t == 0, ZERO_COL, cols + IN_DIM * (t == 2).astype(jnp.int32))
    e = sidx.reshape(N_STRIPE, 16, K_MAX).transpose(0, 2, 1)
    kb = jnp.max(nnz[perm].reshape(N_STRIPE, 16), axis=1).astype(jnp.int32)
    aux = jnp.stack(
        [perm.reshape(N_STRIPE, 16),
         jnp.broadcast_to(kb[:, None], (N_STRIPE, 16))], axis=1)
    return e, aux


def _sc_body(x_hbm, e_hbm, aux_hbm, out_hbm, xb, s, eb, auxb, ob, sem):
    del sem
    wid = lax.axis_index("s") * NC + lax.axis_index("c")
    pltpu.sync_copy(e_hbm, eb)
    pltpu.sync_copy(aux_hbm, auxb)

    def chunk_body(j, carry):
        base = (wid * CHUNKS_PER_W + j) * ROWS_PER_CHUNK
        pltpu.sync_copy(x_hbm.at[pl.ds(base, ROWS_PER_CHUNK)], xb)

        def srow_body(b, c2):
            for c in range(IN_DIM // 16):
                v = xb[b, pl.ds(c * 16, 16)]
                s[pl.ds(b * S_W + c * 16, 16)] = v * 0.5
                e = jnp.exp(v)
                s[pl.ds(b * S_W + IN_DIM + c * 16, 16)] = 1.0 - 2.0 / (e + 1.0)
            s[pl.ds(b * S_W + 2 * IN_DIM, 16)] = jnp.zeros((16,), jnp.float32)
            return c2

        lax.fori_loop(0, ROWS_PER_CHUNK, srow_body, 0)

        zero = jnp.zeros((16,), jnp.float32)
        for r in range(N_STRIPE):
            kmax = jnp.max(auxb[r, 1, :])

            def k_body(k, accs, r=r):
                colv = eb[r, k, :]
                return tuple(
                    accs[b] + plsc.load_gather(s, [colv + b * S_W])
                    for b in range(ROWS_PER_CHUNK)
                )

            accs = lax.fori_loop(
                0, kmax, k_body, tuple(zero for _ in range(ROWS_PER_CHUNK)),
                unroll=False)
            ov = auxb[r, 0, :]
            for b in range(ROWS_PER_CHUNK):
                plsc.store_scatter(
                    ob, [jnp.full((16,), b, jnp.int32), ov], accs[b])

        pltpu.sync_copy(ob, out_hbm.at[pl.ds(base, ROWS_PER_CHUNK)])
        return carry

    lax.fori_loop(0, CHUNKS_PER_W, chunk_body, 0)


@functools.partial(jax.jit, static_argnames=())
def kernel(x, mat):
    e, aux = _build_edges(mat)
    mesh = plsc.VectorSubcoreMesh(core_axis_name="c", subcore_axis_name="s")
    f = functools.partial(
        pl.kernel,
        out_type=jax.ShapeDtypeStruct((BATCH, OUT_DIM), jnp.float32),
        mesh=mesh,
        scratch_types=[
            pltpu.VMEM((ROWS_PER_CHUNK, IN_DIM), jnp.float32),
            pltpu.VMEM((ROWS_PER_CHUNK * S_W,), jnp.float32),
            pltpu.VMEM((N_STRIPE, K_MAX, 16), jnp.int32),
            pltpu.VMEM((N_STRIPE, 2, 16), jnp.int32),
            pltpu.VMEM((ROWS_PER_CHUNK, OUT_DIM), jnp.float32),
            pltpu.SemaphoreType.DMA,
        ],
        compiler_params=pltpu.CompilerParams(
            use_tc_tiling_on_sc=False, needs_layout_passes=False),
    )(_sc_body)
    return f(x, e, aux)
